# baseline (speedup 1.0000x reference)
import jax
import jax.numpy as jnp
from jax.experimental import pallas as pl
from jax.experimental.pallas import tpu as pltpu

_CH = 512
_DEPTH = 4


def _stream_body(x_hbm, p_hbm, o_hbm, pbuf, xbuf, obuf, psem, rsem, wsem):
    n_rows = x_hbm.shape[0]
    n_pos = p_hbm.shape[0]
    n_chunks = n_rows // _CH

    pltpu.make_async_copy(p_hbm, pbuf, psem).start()
    for c in range(min(_DEPTH, n_chunks)):
        pltpu.make_async_copy(
            x_hbm.at[pl.ds(c * _CH, _CH)], xbuf.at[c % _DEPTH], rsem.at[c % _DEPTH]
        ).start()
    pltpu.make_async_copy(p_hbm, pbuf, psem).wait()

    for c in range(n_chunks):
        slot = c % _DEPTH
        pltpu.make_async_copy(
            x_hbm.at[pl.ds(c * _CH, _CH)], xbuf.at[slot], rsem.at[slot]
        ).wait()
        if c >= _DEPTH:
            pltpu.make_async_copy(
                obuf.at[slot], o_hbm.at[pl.ds((c - _DEPTH) * _CH, _CH)], wsem.at[slot]
            ).wait()
        pos_off = (c * _CH) % n_pos
        obuf[slot] = xbuf[slot] + pbuf[pl.ds(pos_off, _CH)]
        pltpu.make_async_copy(
            obuf.at[slot], o_hbm.at[pl.ds(c * _CH, _CH)], wsem.at[slot]
        ).start()
        nxt = c + _DEPTH
        if nxt < n_chunks:
            pltpu.make_async_copy(
                x_hbm.at[pl.ds(nxt * _CH, _CH)], xbuf.at[slot], rsem.at[slot]
            ).start()

    for c in range(max(n_chunks - _DEPTH, 0), n_chunks):
        slot = c % _DEPTH
        pltpu.make_async_copy(
            obuf.at[slot], o_hbm.at[pl.ds(c * _CH, _CH)], wsem.at[slot]
        ).wait()


def kernel(x, pos_table):
    B, S, D = x.shape
    xf = x.reshape(B * S, D)
    out = pl.pallas_call(
        _stream_body,
        in_specs=[
            pl.BlockSpec(memory_space=pltpu.MemorySpace.HBM),
            pl.BlockSpec(memory_space=pltpu.MemorySpace.HBM),
        ],
        out_specs=pl.BlockSpec(memory_space=pltpu.MemorySpace.HBM),
        out_shape=jax.ShapeDtypeStruct((B * S, D), x.dtype),
        scratch_shapes=[
            pltpu.VMEM((S, D), x.dtype),
            pltpu.VMEM((_DEPTH, _CH, D), x.dtype),
            pltpu.VMEM((_DEPTH, _CH, D), x.dtype),
            pltpu.SemaphoreType.DMA,
            pltpu.SemaphoreType.DMA((_DEPTH,)),
            pltpu.SemaphoreType.DMA((_DEPTH,)),
        ],
    )(xf, pos_table[:S])
    return out.reshape(B, S, D)
